# Initial kernel scaffold; baseline (speedup 1.0000x reference)
import jax
import jax.numpy as jnp
from jax import lax
from jax.experimental import pallas as pl
from jax.experimental.pallas import tpu as pltpu


def _tc_body(start_ref, num_ref, lhs_hbm, out_ref, win, sem):
    b = pl.program_id(0)
    s = start_ref[b]
    n = num_ref[b]
    cp = pltpu.make_async_copy(lhs_hbm.at[b, pl.ds(s, 512), :], win, sem)
    cp.start()
    out_ref[0, 256:, :] = jnp.zeros((768, 2048), jnp.float32)
    cp.wait()
    mask = lax.broadcasted_iota(jnp.int32, (256, 1), 0) < n
    out_ref[0, 0:256, 0:1024] = jnp.where(mask, win[0:256, :], 0.0)
    out_ref[0, 0:256, 1024:2048] = jnp.where(mask, win[pl.ds(n, 256), :], 0.0)


def kernel(last_hidden_state, start_marker_indices, num_marker_pairs):
    starts = start_marker_indices.astype(jnp.int32)
    nums = num_marker_pairs.astype(jnp.int32)
    return pl.pallas_call(
        _tc_body,
        grid=(8,),
        in_specs=[
            pl.BlockSpec(memory_space=pltpu.SMEM),
            pl.BlockSpec(memory_space=pltpu.SMEM),
            pl.BlockSpec(memory_space=pltpu.ANY),
        ],
        out_specs=pl.BlockSpec((1, 1024, 2048), lambda b: (b, 0, 0)),
        out_shape=jax.ShapeDtypeStruct((8, 1024, 2048), jnp.float32),
        scratch_shapes=[
            pltpu.VMEM((512, 1024), jnp.float32),
            pltpu.SemaphoreType.DMA,
        ],
    )(starts, nums, last_hidden_state)


# speedup vs baseline: 3.0593x; 3.0593x over previous
import jax
import jax.numpy as jnp
from jax import lax
from jax.experimental import pallas as pl
from jax.experimental.pallas import tpu as pltpu


def _tc_body(start_ref, num_ref, lhs_hbm, out_ref, win, sem):
    b = pl.program_id(0)
    s = start_ref[b]
    n = num_ref[b]
    s0 = (s // 8) * 8
    off = s - s0
    cp = pltpu.make_async_copy(lhs_hbm.at[b, pl.ds(s0, 520), :], win, sem)
    cp.start()
    out_ref[0, 256:, :] = jnp.zeros((768, 2048), jnp.float32)
    cp.wait()
    mask = lax.broadcasted_iota(jnp.int32, (256, 1), 0) < n
    w = win[:]
    first = pltpu.roll(w, 520 - off, 0)[0:256, :]
    second = pltpu.roll(w, 520 - off - n, 0)[0:256, :]
    out_ref[0, 0:256, 0:1024] = jnp.where(mask, first, 0.0)
    out_ref[0, 0:256, 1024:2048] = jnp.where(mask, second, 0.0)


def kernel(last_hidden_state, start_marker_indices, num_marker_pairs):
    starts = start_marker_indices.astype(jnp.int32)
    nums = num_marker_pairs.astype(jnp.int32)
    return pl.pallas_call(
        _tc_body,
        grid=(8,),
        in_specs=[
            pl.BlockSpec(memory_space=pltpu.SMEM),
            pl.BlockSpec(memory_space=pltpu.SMEM),
            pl.BlockSpec(memory_space=pl.ANY),
        ],
        out_specs=pl.BlockSpec((1, 1024, 2048), lambda b: (b, 0, 0)),
        out_shape=jax.ShapeDtypeStruct((8, 1024, 2048), jnp.float32),
        scratch_shapes=[
            pltpu.VMEM((520, 1024), jnp.float32),
            pltpu.SemaphoreType.DMA,
        ],
    )(starts, nums, last_hidden_state)
